# Initial kernel scaffold; baseline (speedup 1.0000x reference)
import jax
import jax.numpy as jnp
from jax.experimental import pallas as pl


def kernel(x, edge_index, W1, b1, g1, be1, W2, b2, g2, be2, W3, b3, W4, b4):
    raise NotImplementedError("write your pallas kernel here")


# speedup vs baseline: 4.4964x; 4.4964x over previous
import functools

import jax
import jax.numpy as jnp
from jax import lax
from jax.experimental import pallas as pl
from jax.experimental.pallas import tpu as pltpu
from jax.experimental.pallas import tpu_sc as plsc

N = 10000
D = 128
H = 256
E = 320000

NP = 10240
EP = 327680
EB = 160
BL = 128

HALF = 5120
AROWS = 5248
JUNK = 5200
ZR = AROWS // 16

RB = 512
GRID = NP // RB
FN = float(N)

_mesh = plsc.VectorSubcoreMesh(core_axis_name="c", subcore_axis_name="s")


@functools.partial(
    pl.kernel,
    out_type=jax.ShapeDtypeStruct((2, HALF, 16), jnp.float32),
    mesh=_mesh,
    scratch_types=[
        pltpu.VMEM((EB, BL), jnp.int32),
        pltpu.VMEM((BL, 16), jnp.float32),
        pltpu.VMEM_SHARED((AROWS, 16), jnp.float32),
        pltpu.SemaphoreType.DMA,
    ],
)
def _deg_kernel(dstr_hbm, ones_hbm, zeros16_hbm, out_hbm, dst_v, ones_v, acc, sem):
    cid = lax.axis_index("c")
    sid = lax.axis_index("s")
    wid = cid * 16 + sid
    pltpu.sync_copy(zeros16_hbm.at[pl.ds(sid * ZR, ZR)],
                    acc.at[pl.ds(sid * ZR, ZR)])
    pltpu.sync_copy(dstr_hbm.at[wid], dst_v)
    pltpu.sync_copy(ones_hbm, ones_v)
    plsc.subcore_barrier()

    @pl.loop(0, EB)
    def _(j):
        pltpu.sync_copy(ones_v, acc.at[dst_v.at[j]], add=True)

    plsc.subcore_barrier()

    @pl.when(sid == 0)
    def _():
        pltpu.sync_copy(acc.at[pl.ds(0, HALF)], out_hbm.at[cid])


@functools.partial(
    pl.kernel,
    out_type=jax.ShapeDtypeStruct((2, HALF, 128), jnp.float32),
    mesh=_mesh,
    scratch_types=[
        pltpu.VMEM((EB, BL), jnp.int32),
        pltpu.VMEM((EB, BL), jnp.int32),
        pltpu.VMEM((BL, 128), jnp.float32),
        pltpu.VMEM((BL, 128), jnp.float32),
        pltpu.VMEM_SHARED((AROWS, 128), jnp.float32),
        pltpu.SemaphoreType.DMA,
        pltpu.SemaphoreType.DMA,
    ],
)
def _agg_kernel(tab_hbm, src_hbm, dstr_hbm, zeros_hbm, out_hbm,
                src_v, dst_v, rows_a, rows_b, acc, sem_a, sem_b):
    cid = lax.axis_index("c")
    sid = lax.axis_index("s")
    wid = cid * 16 + sid
    pltpu.sync_copy(zeros_hbm.at[pl.ds(sid * ZR, ZR)],
                    acc.at[pl.ds(sid * ZR, ZR)])
    pltpu.sync_copy(src_hbm.at[sid], src_v)
    pltpu.sync_copy(dstr_hbm.at[wid], dst_v)
    plsc.subcore_barrier()

    pltpu.async_copy(tab_hbm.at[src_v.at[0]], rows_a, sem_a)

    @pl.loop(0, EB // 2 - 1)
    def _(i):
        j = 2 * i
        gb = pltpu.async_copy(tab_hbm.at[src_v.at[j + 1]], rows_b, sem_b)
        pltpu.make_async_copy(tab_hbm.at[src_v.at[j]], rows_a, sem_a).wait()
        pltpu.sync_copy(rows_a, acc.at[dst_v.at[j]], add=True)
        pltpu.async_copy(tab_hbm.at[src_v.at[j + 2]], rows_a, sem_a)
        gb.wait()
        pltpu.sync_copy(rows_b, acc.at[dst_v.at[j + 1]], add=True)

    gb = pltpu.async_copy(tab_hbm.at[src_v.at[EB - 1]], rows_b, sem_b)
    pltpu.make_async_copy(tab_hbm.at[src_v.at[EB - 2]], rows_a, sem_a).wait()
    pltpu.sync_copy(rows_a, acc.at[dst_v.at[EB - 2]], add=True)
    gb.wait()
    pltpu.sync_copy(rows_b, acc.at[dst_v.at[EB - 1]], add=True)

    plsc.subcore_barrier()

    @pl.when(sid == 0)
    def _():
        pltpu.sync_copy(acc.at[pl.ds(0, HALF)], out_hbm.at[cid])


def _remap_body(d_ref, d0_ref, d1_ref):
    d = d_ref[...]
    d0_ref[...] = jnp.where(d < HALF, d, JUNK)
    d1_ref[...] = jnp.where(d >= HALF, d - HALF, JUNK)


def _remap_call(dst2d):
    return pl.pallas_call(
        _remap_body,
        grid=(5,),
        in_specs=[pl.BlockSpec((RB, 128), lambda i: (i, 0))],
        out_specs=[pl.BlockSpec((RB, 128), lambda i: (i, 0)),
                   pl.BlockSpec((RB, 128), lambda i: (i, 0))],
        out_shape=[jax.ShapeDtypeStruct((EP // 128, 128), jnp.int32),
                   jax.ShapeDtypeStruct((EP // 128, 128), jnp.int32)],
    )(dst2d)


def _scale_body(x_ref, deg_ref, xs_ref, dinvb_ref):
    deg = deg_ref[0, :, 0:1] + 1.0
    dinv = lax.rsqrt(deg)
    dinvb = jnp.broadcast_to(dinv, (RB, 128))
    dinvb_ref[...] = dinvb
    xs_ref[...] = x_ref[...] * dinvb


def _scale_call(x_pad, degp):
    return pl.pallas_call(
        _scale_body,
        grid=(GRID,),
        in_specs=[
            pl.BlockSpec((RB, 128), lambda i: (i, 0)),
            pl.BlockSpec((1, RB, 16), lambda i: (i // 10, i % 10, 0)),
        ],
        out_specs=[
            pl.BlockSpec((RB, 128), lambda i: (i, 0)),
            pl.BlockSpec((RB, 128), lambda i: (i, 0)),
        ],
        out_shape=[
            jax.ShapeDtypeStruct((NP, 128), jnp.float32),
            jax.ShapeDtypeStruct((NP, 128), jnp.float32),
        ],
    )(x_pad, degp)


def _mm1_body(t_ref, xs_ref, dinvb_ref, w_ref, b_ref, y_ref, s_ref, ss_ref):
    i = pl.program_id(0)
    pre = dinvb_ref[...] * (t_ref[0] + xs_ref[...])
    y = jnp.dot(pre, w_ref[...], preferred_element_type=jnp.float32) + b_ref[...]
    y_ref[...] = y
    rows = i * RB + lax.broadcasted_iota(jnp.int32, (RB, 1), 0)
    ym = jnp.where(rows < N, y, 0.0)

    @pl.when(i == 0)
    def _():
        s_ref[...] = jnp.zeros_like(s_ref)
        ss_ref[...] = jnp.zeros_like(ss_ref)

    s_ref[...] += jnp.sum(ym, axis=0, keepdims=True)
    ss_ref[...] += jnp.sum(ym * ym, axis=0, keepdims=True)


def _mm1_call(t1, xs, dinvb, w1, b1):
    return pl.pallas_call(
        _mm1_body,
        grid=(GRID,),
        in_specs=[
            pl.BlockSpec((1, RB, 128), lambda i: (i // 10, i % 10, 0)),
            pl.BlockSpec((RB, 128), lambda i: (i, 0)),
            pl.BlockSpec((RB, 128), lambda i: (i, 0)),
            pl.BlockSpec((128, H), lambda i: (0, 0)),
            pl.BlockSpec((1, H), lambda i: (0, 0)),
        ],
        out_specs=[
            pl.BlockSpec((RB, H), lambda i: (i, 0)),
            pl.BlockSpec((1, H), lambda i: (0, 0)),
            pl.BlockSpec((1, H), lambda i: (0, 0)),
        ],
        out_shape=[
            jax.ShapeDtypeStruct((NP, H), jnp.float32),
            jax.ShapeDtypeStruct((1, H), jnp.float32),
            jax.ShapeDtypeStruct((1, H), jnp.float32),
        ],
    )(t1, xs, dinvb, w1, b1)


def _bn1_body(y_ref, s_ref, ss_ref, g_ref, be_ref, dinvb_ref, ha_ref, hb_ref):
    m = s_ref[...] / FN
    v = ss_ref[...] / FN - m * m
    inv = g_ref[...] * lax.rsqrt(v + 1e-5)
    h = (y_ref[...] - m) * inv + be_ref[...]
    h = jnp.maximum(h, 0.0)
    dinvb = dinvb_ref[...]
    ha_ref[...] = h[:, :128] * dinvb
    hb_ref[...] = h[:, 128:] * dinvb


def _bn1_call(y1, s1, ss1, g1, be1, dinvb):
    return pl.pallas_call(
        _bn1_body,
        grid=(GRID,),
        in_specs=[
            pl.BlockSpec((RB, H), lambda i: (i, 0)),
            pl.BlockSpec((1, H), lambda i: (0, 0)),
            pl.BlockSpec((1, H), lambda i: (0, 0)),
            pl.BlockSpec((1, H), lambda i: (0, 0)),
            pl.BlockSpec((1, H), lambda i: (0, 0)),
            pl.BlockSpec((RB, 128), lambda i: (i, 0)),
        ],
        out_specs=[
            pl.BlockSpec((RB, 128), lambda i: (i, 0)),
            pl.BlockSpec((RB, 128), lambda i: (i, 0)),
        ],
        out_shape=[
            jax.ShapeDtypeStruct((NP, 128), jnp.float32),
            jax.ShapeDtypeStruct((NP, 128), jnp.float32),
        ],
    )(y1, s1, ss1, g1, be1, dinvb)


def _mm2_body(ta_ref, tb_ref, ha_ref, hb_ref, dinvb_ref, w_ref, b_ref,
              y_ref, s_ref, ss_ref):
    i = pl.program_id(0)
    dinvb = dinvb_ref[...]
    prea = dinvb * (ta_ref[0] + ha_ref[...])
    preb = dinvb * (tb_ref[0] + hb_ref[...])
    y = (jnp.dot(prea, w_ref[:128, :], preferred_element_type=jnp.float32)
         + jnp.dot(preb, w_ref[128:, :], preferred_element_type=jnp.float32)
         + b_ref[...])
    y_ref[...] = y
    rows = i * RB + lax.broadcasted_iota(jnp.int32, (RB, 1), 0)
    ym = jnp.where(rows < N, y, 0.0)

    @pl.when(i == 0)
    def _():
        s_ref[...] = jnp.zeros_like(s_ref)
        ss_ref[...] = jnp.zeros_like(ss_ref)

    s_ref[...] += jnp.sum(ym, axis=0, keepdims=True)
    ss_ref[...] += jnp.sum(ym * ym, axis=0, keepdims=True)


def _mm2_call(ta, tb, ha, hb, dinvb, w2, b2):
    return pl.pallas_call(
        _mm2_body,
        grid=(GRID,),
        in_specs=[
            pl.BlockSpec((1, RB, 128), lambda i: (i // 10, i % 10, 0)),
            pl.BlockSpec((1, RB, 128), lambda i: (i // 10, i % 10, 0)),
            pl.BlockSpec((RB, 128), lambda i: (i, 0)),
            pl.BlockSpec((RB, 128), lambda i: (i, 0)),
            pl.BlockSpec((RB, 128), lambda i: (i, 0)),
            pl.BlockSpec((H, H), lambda i: (0, 0)),
            pl.BlockSpec((1, H), lambda i: (0, 0)),
        ],
        out_specs=[
            pl.BlockSpec((RB, H), lambda i: (i, 0)),
            pl.BlockSpec((1, H), lambda i: (0, 0)),
            pl.BlockSpec((1, H), lambda i: (0, 0)),
        ],
        out_shape=[
            jax.ShapeDtypeStruct((NP, H), jnp.float32),
            jax.ShapeDtypeStruct((1, H), jnp.float32),
            jax.ShapeDtypeStruct((1, H), jnp.float32),
        ],
    )(ta, tb, ha, hb, dinvb, w2, b2)


def _head_body(y_ref, s_ref, ss_ref, g_ref, be_ref, w3_ref, b3_ref,
               w4_ref, b4_ref, cs_ref, o_ref):
    i = pl.program_id(0)
    m = s_ref[...] / FN
    v = ss_ref[...] / FN - m * m
    inv = g_ref[...] * lax.rsqrt(v + 1e-5)
    h = (y_ref[...] - m) * inv + be_ref[...]
    h = jnp.maximum(h, 0.0)
    rows = i * RB + lax.broadcasted_iota(jnp.int32, (RB, 1), 0)
    h = jnp.where(rows < N, h, 0.0)

    @pl.when(i == 0)
    def _():
        cs_ref[...] = jnp.zeros_like(cs_ref)

    cs_ref[...] += jnp.sum(h, axis=0, keepdims=True)

    @pl.when(i == GRID - 1)
    def _():
        p = cs_ref[...] / FN
        z = jnp.dot(p, w3_ref[...], preferred_element_type=jnp.float32) + b3_ref[...]
        z = jnp.maximum(z, 0.0)
        o_ref[...] = jnp.dot(z, w4_ref[...], preferred_element_type=jnp.float32) + b4_ref[...]


def _head_call(y2, s2, ss2, g2, be2, w3, b3, w4p, b4p):
    return pl.pallas_call(
        _head_body,
        grid=(GRID,),
        in_specs=[
            pl.BlockSpec((RB, H), lambda i: (i, 0)),
            pl.BlockSpec((1, H), lambda i: (0, 0)),
            pl.BlockSpec((1, H), lambda i: (0, 0)),
            pl.BlockSpec((1, H), lambda i: (0, 0)),
            pl.BlockSpec((1, H), lambda i: (0, 0)),
            pl.BlockSpec((H, H), lambda i: (0, 0)),
            pl.BlockSpec((1, H), lambda i: (0, 0)),
            pl.BlockSpec((H, 128), lambda i: (0, 0)),
            pl.BlockSpec((1, 128), lambda i: (0, 0)),
        ],
        out_specs=[
            pl.BlockSpec((1, H), lambda i: (0, 0)),
            pl.BlockSpec((1, 128), lambda i: (0, 0)),
        ],
        out_shape=[
            jax.ShapeDtypeStruct((1, H), jnp.float32),
            jax.ShapeDtypeStruct((1, 128), jnp.float32),
        ],
    )(y2, s2, ss2, g2, be2, w3, b3, w4p, b4p)


def kernel(x, edge_index, W1, b1, g1, be1, W2, b2, g2, be2, W3, b3, W4, b4):
    pad_e = EP - E
    src = jnp.concatenate([edge_index[0], jnp.full((pad_e,), N, jnp.int32)])
    dst = jnp.concatenate([edge_index[1], jnp.full((pad_e,), N, jnp.int32)])
    src = src.reshape(16, EB, BL)
    d0, d1 = _remap_call(dst.reshape(EP // 128, 128))
    dstr = jnp.concatenate([d0.reshape(16, EB, BL), d1.reshape(16, EB, BL)])

    x_pad = jnp.zeros((NP, 128), jnp.float32).at[:N].set(x)
    zeros_tab = jnp.zeros((AROWS, 128), jnp.float32)
    zeros16 = jnp.zeros((AROWS, 16), jnp.float32)
    ones16 = jnp.ones((BL, 16), jnp.float32)
    b1r = b1.reshape(1, H)
    b2r = b2.reshape(1, H)
    g1r = g1.reshape(1, H)
    g2r = g2.reshape(1, H)
    be1r = be1.reshape(1, H)
    be2r = be2.reshape(1, H)
    b3r = b3.reshape(1, H)
    w4p = jnp.zeros((H, 128), jnp.float32).at[:, 0:1].set(W4)
    b4p = jnp.broadcast_to(b4.reshape(1, 1), (1, 128))

    degp = _deg_kernel(dstr, ones16, zeros16)
    xs, dinvb = _scale_call(x_pad, degp)

    t1 = _agg_kernel(xs, src, dstr, zeros_tab)
    y1, s1, ss1 = _mm1_call(t1, xs, dinvb, W1, b1r)
    ha, hb = _bn1_call(y1, s1, ss1, g1r, be1r, dinvb)

    t2a = _agg_kernel(ha, src, dstr, zeros_tab)
    t2b = _agg_kernel(hb, src, dstr, zeros_tab)
    y2, s2, ss2 = _mm2_call(t2a, t2b, ha, hb, dinvb, W2, b2r)
    _, o = _head_call(y2, s2, ss2, g2r, be2r, W3, b3r, w4p, b4p)
    return o[0, 0:1]
